# Initial kernel scaffold; baseline (speedup 1.0000x reference)
import jax
import jax.numpy as jnp
from jax.experimental import pallas as pl


def kernel(x, edge_index, pe, We1, be1, We2, be2, Wc0, bc0, Wc1, bc1, Wc2, bc2, Wd1, bd1, Wd2, bd2):
    raise NotImplementedError("write your pallas kernel here")


# speedup vs baseline: 6.3012x; 6.3012x over previous
import functools

import jax
import jax.numpy as jnp
from jax import lax
from jax.experimental import pallas as pl
from jax.experimental.pallas import tpu as pltpu
from jax.experimental.pallas import tpu_sc as plsc

N = 100000
E = 1600000
H = 32

NSC = 2
NTILES = 16
HALF = N // NSC
AGG_ROWS = 50176
TRASH = HALF
ZR = 64
EPT = E // NTILES
K = 80
NCHUNK = EPT // K

_mesh = plsc.VectorSubcoreMesh(core_axis_name="c", subcore_axis_name="s")


def _fill(ref, rows, value):
    width = ref.shape[1]
    v = jnp.full((16,), value, jnp.float32)

    def body(i, carry):
        for j in range(width // 16):
            ref[i, pl.ds(j * 16, 16)] = v
        return carry

    lax.fori_loop(0, rows, body, 0)


def _local_idx(dst_v, idx_v, base_node):
    for j in range(K // 16):
        d = dst_v[pl.ds(j * 16, 16)]
        local = d - base_node
        ok = (local >= 0) & (local < HALF)
        idx_v[pl.ds(j * 16, 16)] = jnp.where(ok, local, TRASH)


DROWS = 3128
DROWS_LAST = HALF - 15 * DROWS


def _dump(buf, hbm, c, s):
    @pl.when(s < 15)
    def _():
        pltpu.sync_copy(
            buf.at[pl.ds(s * DROWS, DROWS)],
            hbm.at[pl.ds(c * HALF + s * DROWS, DROWS)],
        )

    @pl.when(s == 15)
    def _():
        pltpu.sync_copy(
            buf.at[pl.ds(15 * DROWS, DROWS_LAST)],
            hbm.at[pl.ds(c * HALF + 15 * DROWS, DROWS_LAST)],
        )


@functools.partial(
    pl.kernel,
    out_type=jax.ShapeDtypeStruct((N, 16), jnp.float32),
    mesh=_mesh,
    compiler_params=pltpu.CompilerParams(use_tc_tiling_on_sc=False),
    scratch_types=[
        pltpu.VMEM((K,), jnp.int32),
        pltpu.VMEM((K,), jnp.int32),
        pltpu.VMEM((K, 16), jnp.float32),
        pltpu.VMEM((ZR, 16), jnp.float32),
        pltpu.VMEM_SHARED((AGG_ROWS, 16), jnp.float32),
    ],
)
def _sc_degree(dst_hbm, deg_hbm, dst_v, idx_v, ones_v, zero_v, hist):
    c = lax.axis_index("c")
    s = lax.axis_index("s")
    base_node = c * HALF

    _fill(zero_v, ZR, 0.0)
    _fill(ones_v, K, 1.0)
    row0 = s * (AGG_ROWS // NTILES)

    def zbody(b, carry):
        pltpu.sync_copy(zero_v, hist.at[pl.ds(row0 + b * ZR, ZR)])
        return carry

    lax.fori_loop(0, AGG_ROWS // NTILES // ZR, zbody, 0)
    plsc.subcore_barrier()

    ebase = s * EPT

    def body(i, carry):
        off = ebase + i * K
        pltpu.sync_copy(dst_hbm.at[pl.ds(off, K)], dst_v)
        _local_idx(dst_v, idx_v, base_node)
        pltpu.sync_copy(ones_v, hist.at[idx_v], add=True)
        return carry

    lax.fori_loop(0, NCHUNK, body, 0)
    plsc.subcore_barrier()

    _dump(hist, deg_hbm, c, s)


@functools.partial(
    pl.kernel,
    out_type=jax.ShapeDtypeStruct((N, H), jnp.float32),
    mesh=_mesh,
    compiler_params=pltpu.CompilerParams(use_tc_tiling_on_sc=False),
    scratch_types=[
        pltpu.VMEM((K,), jnp.int32),
        pltpu.VMEM((K,), jnp.int32),
        pltpu.VMEM((K,), jnp.int32),
        pltpu.VMEM((K, H), jnp.float32),
        pltpu.VMEM((ZR, H), jnp.float32),
        pltpu.VMEM_SHARED((AGG_ROWS, H), jnp.float32),
        pltpu.SemaphoreType.DMA,
    ],
)
def _sc_aggregate(g_hbm, src_hbm, dst_hbm, out_hbm,
                  src_v, dst_v, idx_v, rows_v, zero_v, agg, sem):
    c = lax.axis_index("c")
    s = lax.axis_index("s")
    base_node = c * HALF

    _fill(zero_v, ZR, 0.0)
    row0 = s * (AGG_ROWS // NTILES)

    def zbody(b, carry):
        pltpu.sync_copy(zero_v, agg.at[pl.ds(row0 + b * ZR, ZR)])
        return carry

    lax.fori_loop(0, AGG_ROWS // NTILES // ZR, zbody, 0)
    plsc.subcore_barrier()

    ebase = s * EPT

    def body(i, carry):
        off = ebase + i * K
        pltpu.sync_copy(src_hbm.at[pl.ds(off, K)], src_v)
        pltpu.sync_copy(dst_hbm.at[pl.ds(off, K)], dst_v)
        _local_idx(dst_v, idx_v, base_node)
        pltpu.async_copy(g_hbm.at[src_v], rows_v, sem).wait()
        pltpu.sync_copy(rows_v, agg.at[idx_v], add=True)
        return carry

    lax.fori_loop(0, NCHUNK, body, 0)
    plsc.subcore_barrier()

    _dump(agg, out_hbm, c, s)


BN = 2000
GRID = N // BN


def _row_spec(width):
    return pl.BlockSpec((BN, width), lambda i: (i, 0))


def _full_spec(shape):
    return pl.BlockSpec(shape, lambda i: tuple(0 for _ in shape))


def _dinv32(deg_ref):
    d = jnp.concatenate([deg_ref[...], deg_ref[...]], axis=1) + 1.0
    return lax.rsqrt(d)


def _tc_encoder_body(x_ref, pe_ref, we1_ref, be1_ref, we2_ref, be2_ref, h_ref):
    h = jnp.concatenate([x_ref[...], pe_ref[...]], axis=1)
    a = jnp.maximum(
        jnp.dot(h, we1_ref[...], preferred_element_type=jnp.float32)
        + be1_ref[...], 0.0)
    h_ref[...] = (
        jnp.dot(a, we2_ref[...], preferred_element_type=jnp.float32)
        + be2_ref[...])


def _tc_first_g_body(h_ref, deg_ref, w_ref, g_ref):
    dinv = _dinv32(deg_ref)
    g_ref[...] = dinv * jnp.dot(
        h_ref[...], w_ref[...], preferred_element_type=jnp.float32)


def _tc_mid_body(a_ref, g_ref, deg_ref, w_ref, b_ref, gn_ref):
    dinv = _dinv32(deg_ref)
    h = jnp.maximum(dinv * (a_ref[...] + g_ref[...]) + b_ref[...], 0.0)
    gn_ref[...] = dinv * jnp.dot(
        h, w_ref[...], preferred_element_type=jnp.float32)


def _tc_final_body(a_ref, g_ref, deg_ref, bc_ref, wd1_ref, bd1_ref,
                   wd2_ref, bd2_ref, out_ref):
    dinv = _dinv32(deg_ref)
    h = jnp.maximum(dinv * (a_ref[...] + g_ref[...]) + bc_ref[...], 0.0)
    h = jnp.maximum(
        jnp.dot(h, wd1_ref[...], preferred_element_type=jnp.float32)
        + bd1_ref[...], 0.0)
    out_ref[...] = (
        jnp.dot(h, wd2_ref[...], preferred_element_type=jnp.float32)
        + bd2_ref[...])


def kernel(x, edge_index, pe, We1, be1, We2, be2, Wc0, bc0, Wc1, bc1,
           Wc2, bc2, Wd1, bd1, Wd2, bd2):
    src = edge_index[0]
    dst = edge_index[1]

    deg16 = _sc_degree(dst)

    h0 = pl.pallas_call(
        _tc_encoder_body,
        grid=(GRID,),
        in_specs=[_row_spec(120), _row_spec(8), _full_spec((128, H)),
                  _full_spec((1, H)), _full_spec((H, H)), _full_spec((1, H))],
        out_specs=_row_spec(H),
        out_shape=jax.ShapeDtypeStruct((N, H), jnp.float32),
    )(x, pe, We1, be1.reshape(1, H), We2, be2.reshape(1, H))

    g = pl.pallas_call(
        _tc_first_g_body,
        grid=(GRID,),
        in_specs=[_row_spec(H), _row_spec(16), _full_spec((H, H))],
        out_specs=_row_spec(H),
        out_shape=jax.ShapeDtypeStruct((N, H), jnp.float32),
    )(h0, deg16, Wc0)

    for (w_next, b_cur) in ((Wc1, bc0), (Wc2, bc1)):
        agg = _sc_aggregate(g, src, dst)
        g = pl.pallas_call(
            _tc_mid_body,
            grid=(GRID,),
            in_specs=[_row_spec(H), _row_spec(H), _row_spec(16),
                      _full_spec((H, H)), _full_spec((1, H))],
            out_specs=_row_spec(H),
            out_shape=jax.ShapeDtypeStruct((N, H), jnp.float32),
        )(agg, g, deg16, w_next, b_cur.reshape(1, H))

    agg = _sc_aggregate(g, src, dst)
    out = pl.pallas_call(
        _tc_final_body,
        grid=(GRID,),
        in_specs=[_row_spec(H), _row_spec(H), _row_spec(16),
                  _full_spec((1, H)), _full_spec((H, H)), _full_spec((1, H)),
                  _full_spec((H, 1)), _full_spec((1, 1))],
        out_specs=_row_spec(1),
        out_shape=jax.ShapeDtypeStruct((N, 1), jnp.float32),
    )(agg, g, deg16, bc2.reshape(1, H), Wd1, bd1.reshape(1, H),
      Wd2, bd2.reshape(1, 1))
    return out


# speedup vs baseline: 11.3671x; 1.8040x over previous
import functools

import jax
import jax.numpy as jnp
from jax import lax
from jax.experimental import pallas as pl
from jax.experimental.pallas import tpu as pltpu
from jax.experimental.pallas import tpu_sc as plsc

N = 100000
E = 1600000
H = 32

NSC = 2
NTILES = 16
HALF = N // NSC
AGG_ROWS = 50176
TRASH = HALF
ZR = 64
EPT = E // NTILES
K = 80
NCHUNK = EPT // K

_mesh = plsc.VectorSubcoreMesh(core_axis_name="c", subcore_axis_name="s")


def _fill(ref, rows, value):
    width = ref.shape[1]
    v = jnp.full((16,), value, jnp.float32)

    def body(i, carry):
        for j in range(width // 16):
            ref[i, pl.ds(j * 16, 16)] = v
        return carry

    lax.fori_loop(0, rows, body, 0)


def _local_idx(dst_v, idx_v, base_node):
    for j in range(K // 16):
        d = dst_v[pl.ds(j * 16, 16)]
        local = d - base_node
        ok = (local >= 0) & (local < HALF)
        idx_v[pl.ds(j * 16, 16)] = jnp.where(ok, local, TRASH)


DROWS = 3128
DROWS_LAST = HALF - 15 * DROWS


def _dump(buf, hbm, c, s):
    @pl.when(s < 15)
    def _():
        pltpu.sync_copy(
            buf.at[pl.ds(s * DROWS, DROWS)],
            hbm.at[pl.ds(c * HALF + s * DROWS, DROWS)],
        )

    @pl.when(s == 15)
    def _():
        pltpu.sync_copy(
            buf.at[pl.ds(15 * DROWS, DROWS_LAST)],
            hbm.at[pl.ds(c * HALF + 15 * DROWS, DROWS_LAST)],
        )


@functools.partial(
    pl.kernel,
    out_type=jax.ShapeDtypeStruct((N, 16), jnp.float32),
    mesh=_mesh,
    compiler_params=pltpu.CompilerParams(use_tc_tiling_on_sc=False),
    scratch_types=[
        pltpu.VMEM((K,), jnp.int32),
        pltpu.VMEM((K,), jnp.int32),
        pltpu.VMEM((K, 16), jnp.float32),
        pltpu.VMEM((ZR, 16), jnp.float32),
        pltpu.VMEM_SHARED((AGG_ROWS, 16), jnp.float32),
    ],
)
def _sc_degree(dst_hbm, deg_hbm, dst_v, idx_v, ones_v, zero_v, hist):
    c = lax.axis_index("c")
    s = lax.axis_index("s")
    base_node = c * HALF

    _fill(zero_v, ZR, 0.0)
    _fill(ones_v, K, 1.0)
    row0 = s * (AGG_ROWS // NTILES)

    def zbody(b, carry):
        pltpu.sync_copy(zero_v, hist.at[pl.ds(row0 + b * ZR, ZR)])
        return carry

    lax.fori_loop(0, AGG_ROWS // NTILES // ZR, zbody, 0)
    plsc.subcore_barrier()

    ebase = s * EPT

    def body(i, carry):
        off = ebase + i * K
        pltpu.sync_copy(dst_hbm.at[pl.ds(off, K)], dst_v)
        _local_idx(dst_v, idx_v, base_node)
        pltpu.sync_copy(ones_v, hist.at[idx_v], add=True)
        return carry

    lax.fori_loop(0, NCHUNK, body, 0)
    plsc.subcore_barrier()

    _dump(hist, deg_hbm, c, s)


D = 5
LAG_G = 2
LAG_S = 2


@functools.partial(
    pl.kernel,
    out_type=jax.ShapeDtypeStruct((N, H), jnp.float32),
    mesh=_mesh,
    compiler_params=pltpu.CompilerParams(use_tc_tiling_on_sc=False),
    scratch_types=[
        pltpu.VMEM((D, K), jnp.int32),
        pltpu.VMEM((D, K), jnp.int32),
        pltpu.VMEM((D, K), jnp.int32),
        pltpu.VMEM((D, K, H), jnp.float32),
        pltpu.VMEM((ZR, H), jnp.float32),
        pltpu.VMEM_SHARED((AGG_ROWS, H), jnp.float32),
    ] + [pltpu.SemaphoreType.DMA] * (4 * D),
)
def _sc_aggregate(g_hbm, src_hbm, dst_hbm, out_hbm,
                  sv, dv, sx, rows, zero_v, agg, *sems):
    c = lax.axis_index("c")
    s = lax.axis_index("s")
    base_node = c * HALF
    lsems = sems[0:D]
    lsemd = sems[D:2 * D]
    gsem = sems[2 * D:3 * D]
    ssem = sems[3 * D:4 * D]

    _fill(zero_v, ZR, 0.0)
    row0 = s * (AGG_ROWS // NTILES)

    def zbody(b, carry):
        pltpu.sync_copy(zero_v, agg.at[pl.ds(row0 + b * ZR, ZR)])
        return carry

    lax.fori_loop(0, AGG_ROWS // NTILES // ZR, zbody, 0)
    plsc.subcore_barrier()

    ebase = s * EPT

    def group(gi, carry):
        tick0 = gi * D
        for t in range(D):
            i = tick0 + t

            s1 = t
            off1 = ebase + i * K

            @pl.when(jnp.logical_and(i >= D, i < NCHUNK))
            def _():
                pltpu.make_async_copy(
                    rows.at[s1], agg.at[sx.at[s1]], ssem[s1]).wait()

            @pl.when(i < NCHUNK)
            def _():
                pltpu.async_copy(src_hbm.at[pl.ds(off1, K)], sv.at[s1],
                                 lsems[s1])
                pltpu.async_copy(dst_hbm.at[pl.ds(off1, K)], dv.at[s1],
                                 lsemd[s1])

            c2 = i - LAG_G
            s2 = (t - LAG_G) % D
            off2 = ebase + c2 * K

            @pl.when(jnp.logical_and(c2 >= 0, c2 < NCHUNK))
            def _():
                pltpu.make_async_copy(src_hbm.at[pl.ds(off2, K)], sv.at[s2],
                                      lsems[s2]).wait()
                pltpu.make_async_copy(dst_hbm.at[pl.ds(off2, K)], dv.at[s2],
                                      lsemd[s2]).wait()
                for j in range(K // 16):
                    d = dv[s2, pl.ds(j * 16, 16)]
                    local = d - base_node
                    ok = (local >= 0) & (local < HALF)
                    sx[s2, pl.ds(j * 16, 16)] = jnp.where(ok, local, TRASH)
                pltpu.async_copy(g_hbm.at[sv.at[s2]], rows.at[s2], gsem[s2])

            c3 = i - LAG_G - LAG_S
            s3 = (t - LAG_G - LAG_S) % D

            @pl.when(jnp.logical_and(c3 >= 0, c3 < NCHUNK))
            def _():
                pltpu.make_async_copy(g_hbm.at[sv.at[s3]], rows.at[s3],
                                      gsem[s3]).wait()
                pltpu.async_copy(rows.at[s3], agg.at[sx.at[s3]], ssem[s3],
                                 add=True)
        return carry

    ngroups = (NCHUNK + LAG_G + LAG_S + D - 1) // D + 1
    lax.fori_loop(0, ngroups, group, 0)

    for t in range(D):
        pltpu.make_async_copy(rows.at[t], agg.at[sx.at[t]], ssem[t]).wait()

    plsc.subcore_barrier()

    _dump(agg, out_hbm, c, s)


BN = 2000
GRID = N // BN


def _row_spec(width):
    return pl.BlockSpec((BN, width), lambda i: (i, 0))


def _full_spec(shape):
    return pl.BlockSpec(shape, lambda i: tuple(0 for _ in shape))


def _dinv32(deg_ref):
    d = jnp.concatenate([deg_ref[...], deg_ref[...]], axis=1) + 1.0
    return lax.rsqrt(d)


def _tc_encoder_body(x_ref, pe_ref, we1_ref, be1_ref, we2_ref, be2_ref, h_ref):
    h = jnp.concatenate([x_ref[...], pe_ref[...]], axis=1)
    a = jnp.maximum(
        jnp.dot(h, we1_ref[...], preferred_element_type=jnp.float32)
        + be1_ref[...], 0.0)
    h_ref[...] = (
        jnp.dot(a, we2_ref[...], preferred_element_type=jnp.float32)
        + be2_ref[...])


def _tc_first_g_body(h_ref, deg_ref, w_ref, g_ref):
    dinv = _dinv32(deg_ref)
    g_ref[...] = dinv * jnp.dot(
        h_ref[...], w_ref[...], preferred_element_type=jnp.float32)


def _tc_mid_body(a_ref, g_ref, deg_ref, w_ref, b_ref, gn_ref):
    dinv = _dinv32(deg_ref)
    h = jnp.maximum(dinv * (a_ref[...] + g_ref[...]) + b_ref[...], 0.0)
    gn_ref[...] = dinv * jnp.dot(
        h, w_ref[...], preferred_element_type=jnp.float32)


def _tc_final_body(a_ref, g_ref, deg_ref, bc_ref, wd1_ref, bd1_ref,
                   wd2_ref, bd2_ref, out_ref):
    dinv = _dinv32(deg_ref)
    h = jnp.maximum(dinv * (a_ref[...] + g_ref[...]) + bc_ref[...], 0.0)
    h = jnp.maximum(
        jnp.dot(h, wd1_ref[...], preferred_element_type=jnp.float32)
        + bd1_ref[...], 0.0)
    out_ref[...] = (
        jnp.dot(h, wd2_ref[...], preferred_element_type=jnp.float32)
        + bd2_ref[...])


def kernel(x, edge_index, pe, We1, be1, We2, be2, Wc0, bc0, Wc1, bc1,
           Wc2, bc2, Wd1, bd1, Wd2, bd2):
    src = edge_index[0]
    dst = edge_index[1]

    deg16 = _sc_degree(dst)

    h0 = pl.pallas_call(
        _tc_encoder_body,
        grid=(GRID,),
        in_specs=[_row_spec(120), _row_spec(8), _full_spec((128, H)),
                  _full_spec((1, H)), _full_spec((H, H)), _full_spec((1, H))],
        out_specs=_row_spec(H),
        out_shape=jax.ShapeDtypeStruct((N, H), jnp.float32),
    )(x, pe, We1, be1.reshape(1, H), We2, be2.reshape(1, H))

    g = pl.pallas_call(
        _tc_first_g_body,
        grid=(GRID,),
        in_specs=[_row_spec(H), _row_spec(16), _full_spec((H, H))],
        out_specs=_row_spec(H),
        out_shape=jax.ShapeDtypeStruct((N, H), jnp.float32),
    )(h0, deg16, Wc0)

    for (w_next, b_cur) in ((Wc1, bc0), (Wc2, bc1)):
        agg = _sc_aggregate(g, src, dst)
        g = pl.pallas_call(
            _tc_mid_body,
            grid=(GRID,),
            in_specs=[_row_spec(H), _row_spec(H), _row_spec(16),
                      _full_spec((H, H)), _full_spec((1, H))],
            out_specs=_row_spec(H),
            out_shape=jax.ShapeDtypeStruct((N, H), jnp.float32),
        )(agg, g, deg16, w_next, b_cur.reshape(1, H))

    agg = _sc_aggregate(g, src, dst)
    out = pl.pallas_call(
        _tc_final_body,
        grid=(GRID,),
        in_specs=[_row_spec(H), _row_spec(H), _row_spec(16),
                  _full_spec((1, H)), _full_spec((H, H)), _full_spec((1, H)),
                  _full_spec((H, 1)), _full_spec((1, 1))],
        out_specs=_row_spec(1),
        out_shape=jax.ShapeDtypeStruct((N, 1), jnp.float32),
    )(agg, g, deg16, bc2.reshape(1, H), Wd1, bd1.reshape(1, H),
      Wd2, bd2.reshape(1, 1))
    return out
